# Initial kernel scaffold; baseline (speedup 1.0000x reference)
import jax
import jax.numpy as jnp
from jax.experimental import pallas as pl


def kernel(x, edge_attr, params, edge_index, non_edge_index, batch):
    raise NotImplementedError("write your pallas kernel here")


# speedup vs baseline: 1.0761x; 1.0761x over previous
import functools

import jax
import jax.numpy as jnp
import numpy as np
from jax import lax
from jax.experimental import pallas as pl
from jax.experimental.pallas import tpu as pltpu
from jax.experimental.pallas import tpu_sc as plsc

N = 10000
E = 320000
NE = 320000
G = 64
H = 64
LRELU = 0.01


def _linear_body(x_ref, w_ref, b_ref, o_ref):
    o_ref[...] = (
        jnp.dot(x_ref[...], w_ref[...], preferred_element_type=jnp.float32)
        + b_ref[...]
    )


def _linear(x, w, b, block_m=512):
    m, kin = x.shape
    kout = w.shape[1]
    grid = (pl.cdiv(m, block_m),)
    return pl.pallas_call(
        _linear_body,
        grid=grid,
        in_specs=[
            pl.BlockSpec((block_m, kin), lambda i: (i, 0)),
            pl.BlockSpec((kin, kout), lambda i: (0, 0)),
            pl.BlockSpec((1, kout), lambda i: (0, 0)),
        ],
        out_specs=pl.BlockSpec((block_m, kout), lambda i: (i, 0)),
        out_shape=jax.ShapeDtypeStruct((m, kout), jnp.float32),
    )(x, w, b.reshape(1, kout))


_NC, _NS = 2, 16
_NW = _NC * _NS
_EPW = E // _NW
_CB = 80
_NCHUNK = _EPW // _CB
_NP = 10112
_ROWS = _NP // _NS


def _gen_agg(o, e, src, dst, zeros):
    mesh = plsc.VectorSubcoreMesh(core_axis_name="c", subcore_axis_name="s")

    @functools.partial(
        pl.kernel,
        out_type=jax.ShapeDtypeStruct((_NC, _NP, H), jnp.float32),
        mesh=mesh,
        compiler_params=pltpu.CompilerParams(use_tc_tiling_on_sc=False),
        scratch_types=[
            pltpu.VMEM((_CB,), jnp.int32),
            pltpu.VMEM((_CB,), jnp.int32),
            pltpu.VMEM((_CB, H), jnp.float32),
            pltpu.VMEM((_CB, H), jnp.float32),
            pltpu.VMEM_SHARED((_NP, H), jnp.float32),
            pltpu.SemaphoreType.DMA,
        ],
    )
    def k(o_hbm, e_hbm, src_hbm, dst_hbm, z_hbm, out_hbm,
          sidx, didx, orow, erow, acc, sem):
        cid = lax.axis_index("c")
        sid = lax.axis_index("s")
        wid = sid * _NC + cid
        pltpu.sync_copy(z_hbm.at[pl.ds(sid * _ROWS, _ROWS)],
                        acc.at[pl.ds(sid * _ROWS, _ROWS)])
        plsc.subcore_barrier()

        def chunk(ci, carry):
            base = wid * _EPW + ci * _CB
            pltpu.sync_copy(src_hbm.at[pl.ds(base, _CB)], sidx)
            pltpu.sync_copy(dst_hbm.at[pl.ds(base, _CB)], didx)
            pltpu.sync_copy(e_hbm.at[pl.ds(base, _CB)], erow)
            pltpu.async_copy(o_hbm.at[sidx], orow, sem).wait()

            def row(j, c2):
                def col(cc, c3):
                    sl = pl.ds(cc * 16, 16)
                    t = orow[j, sl] + erow[j, sl]
                    orow[j, sl] = jnp.maximum(t, 0.0) + 1e-7
                    return c3
                return lax.fori_loop(0, H // 16, col, c2)
            lax.fori_loop(0, _CB, row, 0)
            pltpu.sync_copy(orow, acc.at[didx], add=True)
            return carry
        lax.fori_loop(0, _NCHUNK, chunk, 0)
        plsc.subcore_barrier()
        pltpu.sync_copy(acc.at[pl.ds(sid * _ROWS, _ROWS)],
                        out_hbm.at[cid, pl.ds(sid * _ROWS, _ROWS)])

    return k(o, e, src, dst, zeros)


def _alpha_body(q_ref, kj_ref, o_ref):
    o_ref[...] = jnp.sum(q_ref[...] * kj_ref[...], axis=-1) * (1.0 / np.sqrt(H))


def _alpha(qg, kj, block=1024):
    m = qg.shape[0]
    mp = ((m + block - 1) // block) * block
    if mp != m:
        qg = jnp.pad(qg, ((0, mp - m), (0, 0)))
        kj = jnp.pad(kj, ((0, mp - m), (0, 0)))
    out = pl.pallas_call(
        _alpha_body,
        grid=(mp // block,),
        in_specs=[
            pl.BlockSpec((block, H), lambda i: (i, 0)),
            pl.BlockSpec((block, H), lambda i: (i, 0)),
        ],
        out_specs=pl.BlockSpec((block,), lambda i: (i,)),
        out_shape=jax.ShapeDtypeStruct((mp,), jnp.float32),
    )(qg, kj)
    return out[:m]


def _head(h, params, pre):
    t = _linear(h, params[pre + "_w1"], params[pre + "_b1"])
    t = jnp.where(t > 0, t, LRELU * t)
    return _linear(t, params[pre + "_w2"], params[pre + "_b2"])


def kernel(x, edge_attr, params, edge_index, non_edge_index, batch):
    src = edge_index[0]
    dst = edge_index[1]
    o = _linear(x, params["x2h_w"], params["x2h_b"])
    e = _linear(edge_attr, params["e2h_w"], params["e2h_b"])
    scale = 1.0 / np.sqrt(H)
    zb = jnp.zeros((H,), jnp.float32)
    znh = jnp.zeros((_NP, H), jnp.float32)
    for i in range(6):
        msg = jax.nn.relu(o[src] + e) + 1e-7
        agg = jax.ops.segment_sum(msg, dst, num_segments=N)
        o = o + _linear(agg + o, params["gen_w"][i], params["gen_b"][i])
        q = _linear(o, params["tq_w"][i], params["tq_b"][i])
        k = _linear(o, params["tk_w"][i], params["tk_b"][i])
        v = _linear(o, params["tv_w"][i], params["tv_b"][i])
        ee = _linear(e, params["te_w"][i], zb)
        kj = k[src] + ee
        alpha = jnp.sum(q[dst] * kj, axis=-1) * scale
        amax = jax.ops.segment_max(alpha, dst, num_segments=N)
        amax = jnp.where(jnp.isfinite(amax), amax, 0.0)
        ex = jnp.exp(alpha - amax[dst])
        den = jax.ops.segment_sum(ex, dst, num_segments=N)
        a = ex / (den[dst] + 1e-16)
        agg2 = jax.ops.segment_sum((v[src] + ee) * a[:, None], dst, num_segments=N)
        o = o + (agg2 + o @ params["ts_w"][i] + params["ts_b"][i])
    ones = jnp.ones((N,), jnp.float32)
    cnt = jax.ops.segment_sum(ones, batch, num_segments=G)
    gsum = jax.ops.segment_sum(o, batch, num_segments=G)
    glob = gsum / jnp.maximum(cnt, 1.0)[:, None]
    ne_row = non_edge_index[0]
    ne_col = non_edge_index[1]
    e_row = edge_index[0, ::2]
    e_col = edge_index[1, ::2]
    return (
        _head(glob, params, "stop"),
        _head(o, params, "add_node"),
        _head(o, params, "node_attr"),
        _head(o[ne_row] + o[ne_col], params, "add_edge"),
        _head(o[e_row] + o[e_col], params, "edge_attr_h"),
    )


# speedup vs baseline: 1.1129x; 1.0342x over previous
import functools

import jax
import jax.numpy as jnp
import numpy as np
from jax import lax
from jax.experimental import pallas as pl
from jax.experimental.pallas import tpu as pltpu
from jax.experimental.pallas import tpu_sc as plsc

N = 10000
E = 320000
NE = 320000
G = 64
H = 64
LRELU = 0.01


def _linear_body(x_ref, w_ref, b_ref, o_ref):
    o_ref[...] = (
        jnp.dot(x_ref[...], w_ref[...], preferred_element_type=jnp.float32)
        + b_ref[...]
    )


def _linear(x, w, b, block_m=512):
    m, kin = x.shape
    kout = w.shape[1]
    grid = (pl.cdiv(m, block_m),)
    return pl.pallas_call(
        _linear_body,
        grid=grid,
        in_specs=[
            pl.BlockSpec((block_m, kin), lambda i: (i, 0)),
            pl.BlockSpec((kin, kout), lambda i: (0, 0)),
            pl.BlockSpec((1, kout), lambda i: (0, 0)),
        ],
        out_specs=pl.BlockSpec((block_m, kout), lambda i: (i, 0)),
        out_shape=jax.ShapeDtypeStruct((m, kout), jnp.float32),
    )(x, w, b.reshape(1, kout))


_NC, _NS = 2, 16
_NW = _NC * _NS
_EPW = E // _NW
_CB = 80
_NCHUNK = _EPW // _CB
_NP = 10112
_ROWS = _NP // _NS


def _gen_agg(o, e, src, dst, zeros):
    mesh = plsc.VectorSubcoreMesh(core_axis_name="c", subcore_axis_name="s")

    @functools.partial(
        pl.kernel,
        out_type=jax.ShapeDtypeStruct((_NC, _NP, H), jnp.float32),
        mesh=mesh,
        compiler_params=pltpu.CompilerParams(use_tc_tiling_on_sc=False),
        scratch_types=[
            pltpu.VMEM((_CB,), jnp.int32),
            pltpu.VMEM((_CB,), jnp.int32),
            pltpu.VMEM((_CB, H), jnp.float32),
            pltpu.VMEM((_CB, H), jnp.float32),
            pltpu.VMEM_SHARED((_NP, H), jnp.float32),
            pltpu.SemaphoreType.DMA,
        ],
    )
    def k(o_hbm, e_hbm, src_hbm, dst_hbm, z_hbm, out_hbm,
          sidx, didx, orow, erow, acc, sem):
        cid = lax.axis_index("c")
        sid = lax.axis_index("s")
        wid = sid * _NC + cid
        pltpu.sync_copy(z_hbm.at[pl.ds(sid * _ROWS, _ROWS)],
                        acc.at[pl.ds(sid * _ROWS, _ROWS)])
        plsc.subcore_barrier()

        def chunk(ci, carry):
            base = wid * _EPW + ci * _CB
            pltpu.sync_copy(src_hbm.at[pl.ds(base, _CB)], sidx)
            pltpu.sync_copy(dst_hbm.at[pl.ds(base, _CB)], didx)
            pltpu.sync_copy(e_hbm.at[pl.ds(base, _CB)], erow)
            pltpu.async_copy(o_hbm.at[sidx], orow, sem).wait()

            def row(j, c2):
                def col(cc, c3):
                    sl = pl.ds(cc * 16, 16)
                    t = orow[j, sl] + erow[j, sl]
                    orow[j, sl] = jnp.maximum(t, 0.0) + 1e-7
                    return c3
                return lax.fori_loop(0, H // 16, col, c2)
            lax.fori_loop(0, _CB, row, 0)
            pltpu.sync_copy(orow, acc.at[didx], add=True)
            return carry
        lax.fori_loop(0, _NCHUNK, chunk, 0)
        plsc.subcore_barrier()
        pltpu.sync_copy(acc.at[pl.ds(sid * _ROWS, _ROWS)],
                        out_hbm.at[cid, pl.ds(sid * _ROWS, _ROWS)])

    return k(o, e, src, dst, zeros)


def _pair_gather(tab, ridx, cidx, cb):
    m = ridx.shape[0]
    mpw = m // _NW
    nch = mpw // cb
    mesh = plsc.VectorSubcoreMesh(core_axis_name="c", subcore_axis_name="s")

    @functools.partial(
        pl.kernel,
        out_type=jax.ShapeDtypeStruct((m, H), jnp.float32),
        mesh=mesh,
        compiler_params=pltpu.CompilerParams(use_tc_tiling_on_sc=False),
        scratch_types=[
            pltpu.VMEM((cb,), jnp.int32),
            pltpu.VMEM((cb,), jnp.int32),
            pltpu.VMEM((cb, H), jnp.float32),
            pltpu.VMEM((cb, H), jnp.float32),
            pltpu.SemaphoreType.DMA,
            pltpu.SemaphoreType.DMA,
        ],
    )
    def k(tab_hbm, r_hbm, c_hbm, out_hbm, ridx_v, cidx_v, rrow, crow, sem1, sem2):
        cid = lax.axis_index("c")
        sid = lax.axis_index("s")
        wid = sid * _NC + cid

        def chunk(ci, carry):
            base = wid * mpw + ci * cb
            pltpu.sync_copy(r_hbm.at[pl.ds(base, cb)], ridx_v)
            pltpu.sync_copy(c_hbm.at[pl.ds(base, cb)], cidx_v)
            cp1 = pltpu.async_copy(tab_hbm.at[ridx_v], rrow, sem1)
            cp2 = pltpu.async_copy(tab_hbm.at[cidx_v], crow, sem2)
            cp1.wait()
            cp2.wait()

            def row(j, c2):
                def col(cc, c3):
                    sl = pl.ds(cc * 16, 16)
                    rrow[j, sl] = rrow[j, sl] + crow[j, sl]
                    return c3
                return lax.fori_loop(0, H // 16, col, c2)
            lax.fori_loop(0, cb, row, 0)
            pltpu.sync_copy(rrow, out_hbm.at[pl.ds(base, cb)])
            return carry
        lax.fori_loop(0, nch, chunk, 0)

    return k(tab, ridx, cidx)


_PB = 80
_PCHUNK = N // _PB


def _pool(o, batch, zeros_g, zeros_c, ones_n):
    mesh = plsc.VectorSubcoreMesh(core_axis_name="c", subcore_axis_name="s")

    @functools.partial(
        pl.kernel,
        out_type=(jax.ShapeDtypeStruct((_NC, G, H), jnp.float32),
                  jax.ShapeDtypeStruct((_NC, G), jnp.float32)),
        mesh=mesh,
        compiler_params=pltpu.CompilerParams(use_tc_tiling_on_sc=False),
        scratch_types=[
            pltpu.VMEM((_PB,), jnp.int32),
            pltpu.VMEM((_PB, H), jnp.float32),
            pltpu.VMEM((_PB,), jnp.float32),
            pltpu.VMEM_SHARED((G, H), jnp.float32),
            pltpu.VMEM_SHARED((G,), jnp.float32),
        ],
    )
    def k(o_hbm, b_hbm, zg_hbm, zc_hbm, on_hbm, gout_hbm, cout_hbm,
          bidx, orow, ovec, gacc, cacc):
        cid = lax.axis_index("c")
        sid = lax.axis_index("s")
        wid = sid * _NC + cid

        @pl.when(sid == 0)
        def _():
            pltpu.sync_copy(zg_hbm, gacc)
            pltpu.sync_copy(zc_hbm, cacc)
        plsc.subcore_barrier()

        def chunk(ci, carry):
            base = (ci * _NW + wid) * _PB
            pltpu.sync_copy(b_hbm.at[pl.ds(base, _PB)], bidx)
            pltpu.sync_copy(o_hbm.at[pl.ds(base, _PB)], orow)
            pltpu.sync_copy(on_hbm.at[pl.ds(base, _PB)], ovec)
            pltpu.sync_copy(orow, gacc.at[bidx], add=True)
            pltpu.sync_copy(ovec, cacc.at[bidx], add=True)
            return carry
        lax.fori_loop(0, _PCHUNK // _NW, chunk, 0)
        @pl.when(wid < _PCHUNK - (_PCHUNK // _NW) * _NW)
        def _():
            base = ((_PCHUNK // _NW) * _NW + wid) * _PB
            pltpu.sync_copy(b_hbm.at[pl.ds(base, _PB)], bidx)
            pltpu.sync_copy(o_hbm.at[pl.ds(base, _PB)], orow)
            pltpu.sync_copy(on_hbm.at[pl.ds(base, _PB)], ovec)
            pltpu.sync_copy(orow, gacc.at[bidx], add=True)
            pltpu.sync_copy(ovec, cacc.at[bidx], add=True)
        plsc.subcore_barrier()

        @pl.when(sid == 0)
        def _():
            pltpu.sync_copy(gacc, gout_hbm.at[cid])
            pltpu.sync_copy(cacc, cout_hbm.at[cid])

    return k(o, batch, zeros_g, zeros_c, ones_n)


def _alpha_body(q_ref, kj_ref, o_ref):
    o_ref[...] = jnp.sum(q_ref[...] * kj_ref[...], axis=-1) * (1.0 / np.sqrt(H))


def _alpha(qg, kj, block=1024):
    m = qg.shape[0]
    mp = ((m + block - 1) // block) * block
    if mp != m:
        qg = jnp.pad(qg, ((0, mp - m), (0, 0)))
        kj = jnp.pad(kj, ((0, mp - m), (0, 0)))
    out = pl.pallas_call(
        _alpha_body,
        grid=(mp // block,),
        in_specs=[
            pl.BlockSpec((block, H), lambda i: (i, 0)),
            pl.BlockSpec((block, H), lambda i: (i, 0)),
        ],
        out_specs=pl.BlockSpec((block,), lambda i: (i,)),
        out_shape=jax.ShapeDtypeStruct((mp,), jnp.float32),
    )(qg, kj)
    return out[:m]


def _head(h, params, pre):
    t = _linear(h, params[pre + "_w1"], params[pre + "_b1"])
    t = jnp.where(t > 0, t, LRELU * t)
    return _linear(t, params[pre + "_w2"], params[pre + "_b2"])


def kernel(x, edge_attr, params, edge_index, non_edge_index, batch):
    src = edge_index[0]
    dst = edge_index[1]
    o = _linear(x, params["x2h_w"], params["x2h_b"])
    e = _linear(edge_attr, params["e2h_w"], params["e2h_b"])
    scale = 1.0 / np.sqrt(H)
    zb = jnp.zeros((H,), jnp.float32)
    znh = jnp.zeros((_NP, H), jnp.float32)
    for i in range(6):
        msg = jax.nn.relu(o[src] + e) + 1e-7
        agg = jax.ops.segment_sum(msg, dst, num_segments=N)
        o = o + _linear(agg + o, params["gen_w"][i], params["gen_b"][i])
        q = _linear(o, params["tq_w"][i], params["tq_b"][i])
        k = _linear(o, params["tk_w"][i], params["tk_b"][i])
        v = _linear(o, params["tv_w"][i], params["tv_b"][i])
        ee = _linear(e, params["te_w"][i], zb)
        kj = k[src] + ee
        alpha = jnp.sum(q[dst] * kj, axis=-1) * scale
        amax = jax.ops.segment_max(alpha, dst, num_segments=N)
        amax = jnp.where(jnp.isfinite(amax), amax, 0.0)
        ex = jnp.exp(alpha - amax[dst])
        den = jax.ops.segment_sum(ex, dst, num_segments=N)
        a = ex / (den[dst] + 1e-16)
        agg2 = jax.ops.segment_sum((v[src] + ee) * a[:, None], dst, num_segments=N)
        o = o + (agg2 + o @ params["ts_w"][i] + params["ts_b"][i])
    gsum_p, cnt_p = _pool(
        o, batch,
        jnp.zeros((G, H), jnp.float32), jnp.zeros((G,), jnp.float32),
        jnp.ones((N,), jnp.float32),
    )
    cnt = cnt_p[0] + cnt_p[1]
    gsum = gsum_p[0] + gsum_p[1]
    glob = gsum / jnp.maximum(cnt, 1.0)[:, None]
    ne_row = non_edge_index[0]
    ne_col = non_edge_index[1]
    e_row = edge_index[0, ::2] + 0
    e_col = edge_index[1, ::2] + 0
    return (
        _head(glob, params, "stop"),
        _head(o, params, "add_node"),
        _head(o, params, "node_attr"),
        _head(_pair_gather(o, ne_row, ne_col, 80), params, "add_edge"),
        _head(_pair_gather(o, e_row, e_col, 40), params, "edge_attr_h"),
    )


# speedup vs baseline: 1.1129x; 1.0000x over previous
import functools

import jax
import jax.numpy as jnp
import numpy as np
from jax import lax
from jax.experimental import pallas as pl
from jax.experimental.pallas import tpu as pltpu
from jax.experimental.pallas import tpu_sc as plsc

N = 10000
E = 320000
NE = 320000
G = 64
H = 64
LRELU = 0.01


def _linear_body(x_ref, w_ref, b_ref, o_ref):
    o_ref[...] = (
        jnp.dot(x_ref[...], w_ref[...], preferred_element_type=jnp.float32)
        + b_ref[...]
    )


def _linear(x, w, b, block_m=512):
    m, kin = x.shape
    kout = w.shape[1]
    grid = (pl.cdiv(m, block_m),)
    return pl.pallas_call(
        _linear_body,
        grid=grid,
        in_specs=[
            pl.BlockSpec((block_m, kin), lambda i: (i, 0)),
            pl.BlockSpec((kin, kout), lambda i: (0, 0)),
            pl.BlockSpec((1, kout), lambda i: (0, 0)),
        ],
        out_specs=pl.BlockSpec((block_m, kout), lambda i: (i, 0)),
        out_shape=jax.ShapeDtypeStruct((m, kout), jnp.float32),
    )(x, w, b.reshape(1, kout))


_NC, _NS = 2, 16
_NW = _NC * _NS


def _pair_gather(tab, ridx, cidx, cb):
    m = ridx.shape[0]
    mpw = m // _NW
    nch = mpw // cb
    mesh = plsc.VectorSubcoreMesh(core_axis_name="c", subcore_axis_name="s")

    @functools.partial(
        pl.kernel,
        out_type=jax.ShapeDtypeStruct((m, H), jnp.float32),
        mesh=mesh,
        compiler_params=pltpu.CompilerParams(use_tc_tiling_on_sc=False),
        scratch_types=[
            pltpu.VMEM((cb,), jnp.int32),
            pltpu.VMEM((cb,), jnp.int32),
            pltpu.VMEM((cb, H), jnp.float32),
            pltpu.VMEM((cb, H), jnp.float32),
            pltpu.SemaphoreType.DMA,
            pltpu.SemaphoreType.DMA,
        ],
    )
    def k(tab_hbm, r_hbm, c_hbm, out_hbm, ridx_v, cidx_v, rrow, crow, sem1, sem2):
        cid = lax.axis_index("c")
        sid = lax.axis_index("s")
        wid = sid * _NC + cid

        def chunk(ci, carry):
            base = wid * mpw + ci * cb
            pltpu.sync_copy(r_hbm.at[pl.ds(base, cb)], ridx_v)
            pltpu.sync_copy(c_hbm.at[pl.ds(base, cb)], cidx_v)
            cp1 = pltpu.async_copy(tab_hbm.at[ridx_v], rrow, sem1)
            cp2 = pltpu.async_copy(tab_hbm.at[cidx_v], crow, sem2)
            cp1.wait()
            cp2.wait()

            def row(j, c2):
                def col(cc, c3):
                    sl = pl.ds(cc * 16, 16)
                    rrow[j, sl] = rrow[j, sl] + crow[j, sl]
                    return c3
                return lax.fori_loop(0, H // 16, col, c2)
            lax.fori_loop(0, cb, row, 0)
            pltpu.sync_copy(rrow, out_hbm.at[pl.ds(base, cb)])
            return carry
        lax.fori_loop(0, nch, chunk, 0)

    return k(tab, ridx, cidx)


_PB = 80
_PCHUNK = N // _PB


def _pool(o, batch, zeros_g, zeros_c, ones_n):
    mesh = plsc.VectorSubcoreMesh(core_axis_name="c", subcore_axis_name="s")

    @functools.partial(
        pl.kernel,
        out_type=(jax.ShapeDtypeStruct((_NC, G, H), jnp.float32),
                  jax.ShapeDtypeStruct((_NC, G), jnp.float32)),
        mesh=mesh,
        compiler_params=pltpu.CompilerParams(use_tc_tiling_on_sc=False),
        scratch_types=[
            pltpu.VMEM((_PB,), jnp.int32),
            pltpu.VMEM((_PB, H), jnp.float32),
            pltpu.VMEM((_PB,), jnp.float32),
            pltpu.VMEM_SHARED((G, H), jnp.float32),
            pltpu.VMEM_SHARED((G,), jnp.float32),
        ],
    )
    def k(o_hbm, b_hbm, zg_hbm, zc_hbm, on_hbm, gout_hbm, cout_hbm,
          bidx, orow, ovec, gacc, cacc):
        cid = lax.axis_index("c")
        sid = lax.axis_index("s")
        wid = sid * _NC + cid

        @pl.when(sid == 0)
        def _():
            pltpu.sync_copy(zg_hbm, gacc)
            pltpu.sync_copy(zc_hbm, cacc)
        plsc.subcore_barrier()

        def chunk(ci, carry):
            base = (ci * _NW + wid) * _PB
            pltpu.sync_copy(b_hbm.at[pl.ds(base, _PB)], bidx)
            pltpu.sync_copy(o_hbm.at[pl.ds(base, _PB)], orow)
            pltpu.sync_copy(on_hbm.at[pl.ds(base, _PB)], ovec)
            pltpu.sync_copy(orow, gacc.at[bidx], add=True)
            pltpu.sync_copy(ovec, cacc.at[bidx], add=True)
            return carry
        lax.fori_loop(0, _PCHUNK // _NW, chunk, 0)
        @pl.when(wid < _PCHUNK - (_PCHUNK // _NW) * _NW)
        def _():
            base = ((_PCHUNK // _NW) * _NW + wid) * _PB
            pltpu.sync_copy(b_hbm.at[pl.ds(base, _PB)], bidx)
            pltpu.sync_copy(o_hbm.at[pl.ds(base, _PB)], orow)
            pltpu.sync_copy(on_hbm.at[pl.ds(base, _PB)], ovec)
            pltpu.sync_copy(orow, gacc.at[bidx], add=True)
            pltpu.sync_copy(ovec, cacc.at[bidx], add=True)
        plsc.subcore_barrier()

        @pl.when(sid == 0)
        def _():
            pltpu.sync_copy(gacc, gout_hbm.at[cid])
            pltpu.sync_copy(cacc, cout_hbm.at[cid])

    return k(o, batch, zeros_g, zeros_c, ones_n)


def _head(h, params, pre):
    t = _linear(h, params[pre + "_w1"], params[pre + "_b1"])
    t = jnp.where(t > 0, t, LRELU * t)
    return _linear(t, params[pre + "_w2"], params[pre + "_b2"])


def kernel(x, edge_attr, params, edge_index, non_edge_index, batch):
    src = edge_index[0]
    dst = edge_index[1]
    o = _linear(x, params["x2h_w"], params["x2h_b"])
    e = _linear(edge_attr, params["e2h_w"], params["e2h_b"])
    scale = 1.0 / np.sqrt(H)
    zb = jnp.zeros((H,), jnp.float32)
    for i in range(6):
        msg = jax.nn.relu(o[src] + e) + 1e-7
        agg = jax.ops.segment_sum(msg, dst, num_segments=N)
        o = o + _linear(agg + o, params["gen_w"][i], params["gen_b"][i])
        q = _linear(o, params["tq_w"][i], params["tq_b"][i])
        k = _linear(o, params["tk_w"][i], params["tk_b"][i])
        v = _linear(o, params["tv_w"][i], params["tv_b"][i])
        ee = _linear(e, params["te_w"][i], zb)
        kj = k[src] + ee
        alpha = jnp.sum(q[dst] * kj, axis=-1) * scale
        amax = jax.ops.segment_max(alpha, dst, num_segments=N)
        amax = jnp.where(jnp.isfinite(amax), amax, 0.0)
        ex = jnp.exp(alpha - amax[dst])
        den = jax.ops.segment_sum(ex, dst, num_segments=N)
        a = ex / (den[dst] + 1e-16)
        agg2 = jax.ops.segment_sum((v[src] + ee) * a[:, None], dst, num_segments=N)
        o = o + (agg2 + o @ params["ts_w"][i] + params["ts_b"][i])
    gsum_p, cnt_p = _pool(
        o, batch,
        jnp.zeros((G, H), jnp.float32), jnp.zeros((G,), jnp.float32),
        jnp.ones((N,), jnp.float32),
    )
    cnt = cnt_p[0] + cnt_p[1]
    gsum = gsum_p[0] + gsum_p[1]
    glob = gsum / jnp.maximum(cnt, 1.0)[:, None]
    ne_row = non_edge_index[0]
    ne_col = non_edge_index[1]
    e_row = edge_index[0, ::2] + 0
    e_col = edge_index[1, ::2] + 0
    return (
        _head(glob, params, "stop"),
        _head(o, params, "add_node"),
        _head(o, params, "node_attr"),
        _head(_pair_gather(o, ne_row, ne_col, 80), params, "add_edge"),
        _head(_pair_gather(o, e_row, e_col, 40), params, "edge_attr_h"),
    )
